# baseline (speedup 1.0000x reference)
import functools

import jax
import jax.numpy as jnp
from jax import lax
from jax.experimental import pallas as pl
from jax.experimental.pallas import tpu as pltpu
from jax.experimental.pallas import tpu_sc as plsc

N = 1024
CHANNEL = 64
CLASS_NUM = 100000
ALPHA = 0.85
T = 1.0

ROWS2 = CLASS_NUM // 2
BLK = 5000
NBLK = CLASS_NUM // BLK


def _gather_pairs_sc(bank2, label):
    info = plsc.get_sparse_core_info()
    num_workers = info.num_cores * info.num_subcores
    b_per_w = N // num_workers
    mesh = plsc.VectorSubcoreMesh(core_axis_name="c", subcore_axis_name="s")

    @functools.partial(
        pl.kernel,
        mesh=mesh,
        out_type=jax.ShapeDtypeStruct((N, 2 * CHANNEL), jnp.float32),
        scratch_types=[
            pltpu.VMEM((b_per_w,), jnp.int32),
            pltpu.VMEM((b_per_w,), jnp.int32),
            pltpu.VMEM((b_per_w, 2 * CHANNEL), jnp.float32),
            pltpu.SemaphoreType.DMA,
        ],
    )
    def gather_kernel(label_hbm, bank_hbm, out_hbm, idx_v, pidx_v, rows_v,
                      sem):
        wid = lax.axis_index("s") * info.num_cores + lax.axis_index("c")
        base = wid * b_per_w
        pltpu.sync_copy(label_hbm.at[pl.ds(base, b_per_w)], idx_v)
        for h in range(b_per_w // 16):
            lv = idx_v[pl.ds(h * 16, 16)]
            pidx_v[pl.ds(h * 16, 16)] = lax.shift_right_logical(lv, 1)
        pltpu.async_copy(bank_hbm.at[pidx_v], rows_v, sem).wait()
        pltpu.sync_copy(rows_v, out_hbm.at[pl.ds(base, b_per_w)])

    return gather_kernel(label, bank2)


def _loss_body(f_ref, bank_ref, old2_ref, lrow_ref, lcol_ref, out_ref,
               s_ref):
    i = pl.program_id(0)
    f = f_ref[...]
    blk = bank_ref[...]
    dn = (((1,), (1,)), ((), ()))
    s_blk = lax.dot_general(f, blk, dn,
                            preferred_element_type=jnp.float32) / T
    part = jnp.sum(jnp.exp(s_blk), axis=1, keepdims=True)

    @pl.when(i == 0)
    def _init():
        s_ref[...] = part

    @pl.when(i > 0)
    def _accum():
        s_ref[...] = s_ref[...] + part

    @pl.when(i == NBLK - 1)
    def _finish():
        old2 = old2_ref[...]
        lcol = lcol_ref[...]
        odd = lax.rem(lcol, 2).astype(jnp.float32)
        old = (old2[:, :CHANNEL] * (1.0 - odd)
               + old2[:, CHANNEL:] * odd)
        p = lax.dot_general(f, old, dn,
                            preferred_element_type=jnp.float32) / T
        q = lax.dot_general(f, f, dn,
                            preferred_element_type=jnp.float32) / T
        lrow = lrow_ref[...]
        row_i = lax.broadcasted_iota(jnp.int32, (N, N), 0)
        col_i = lax.broadcasted_iota(jnp.int32, (N, N), 1)
        later_same = jnp.logical_and(lcol == lrow, row_i > col_i)
        winner = jnp.logical_not(jnp.any(later_same, axis=0, keepdims=True))
        wmask = jnp.broadcast_to(winner, (N, N))
        newv = ALPHA * p + q
        cmax = jnp.max(jnp.where(wmask, newv, -1e30), axis=1, keepdims=True)
        m_fin = jnp.maximum(cmax, 0.0)
        corr = jnp.sum(
            jnp.where(wmask, jnp.exp(newv - m_fin) - jnp.exp(p - m_fin), 0.0),
            axis=1, keepdims=True)
        total = s_ref[...] * jnp.exp(-m_fin) + corr
        lse = m_fin + jnp.log(total)
        pdiag = jnp.sum(jnp.where(row_i == col_i, p, 0.0), axis=1,
                        keepdims=True)
        same_win = jnp.logical_and(lcol == lrow, wmask)
        qsel = jnp.sum(jnp.where(same_win, q, 0.0), axis=1, keepdims=True)
        pos = ALPHA * pdiag + qsel
        out_ref[...] = jnp.mean(lse - pos, axis=(0, 1), keepdims=True)


def _loss_tc(f_normed, bank, old2, label):
    lrow = label.reshape(1, N)
    lcol = label.reshape(N, 1)
    out = pl.pallas_call(
        _loss_body,
        grid=(NBLK,),
        in_specs=[
            pl.BlockSpec((N, CHANNEL), lambda i: (0, 0)),
            pl.BlockSpec((BLK, CHANNEL), lambda i: (i, 0)),
            pl.BlockSpec((N, 2 * CHANNEL), lambda i: (0, 0)),
            pl.BlockSpec((1, N), lambda i: (0, 0)),
            pl.BlockSpec((N, 1), lambda i: (0, 0)),
        ],
        out_specs=pl.BlockSpec((1, 1), lambda i: (0, 0)),
        out_shape=jax.ShapeDtypeStruct((1, 1), jnp.float32),
        scratch_shapes=[
            pltpu.VMEM((N, 1), jnp.float32),
        ],
    )(f_normed, bank, old2, lrow, lcol)
    return out[0, 0]


def kernel(f_normed, bank, label):
    bank2 = bank.reshape(ROWS2, 2 * CHANNEL)
    old2 = _gather_pairs_sc(bank2, label)
    return _loss_tc(f_normed, bank, old2, label)


# speedup vs baseline: 1.2500x; 1.2500x over previous
import functools

import jax
import jax.numpy as jnp
from jax import lax
from jax.experimental import pallas as pl
from jax.experimental.pallas import tpu as pltpu
from jax.experimental.pallas import tpu_sc as plsc

N = 1024
CHANNEL = 64
CLASS_NUM = 100000
ALPHA = 0.85
T = 1.0

ROWS2 = CLASS_NUM // 2
BLK = 4000
NBLK = CLASS_NUM // BLK


def _gather_pairs_sc(bank2, label):
    info = plsc.get_sparse_core_info()
    num_workers = info.num_cores * info.num_subcores
    b_per_w = N // num_workers
    mesh = plsc.VectorSubcoreMesh(core_axis_name="c", subcore_axis_name="s")

    @functools.partial(
        pl.kernel,
        mesh=mesh,
        out_type=jax.ShapeDtypeStruct((N, 2 * CHANNEL), jnp.float32),
        scratch_types=[
            pltpu.VMEM((b_per_w,), jnp.int32),
            pltpu.VMEM((b_per_w,), jnp.int32),
            pltpu.VMEM((b_per_w, 2 * CHANNEL), jnp.float32),
            pltpu.SemaphoreType.DMA,
        ],
        compiler_params=pltpu.CompilerParams(use_tc_tiling_on_sc=True),
    )
    def gather_kernel(label_hbm, bank_hbm, out_hbm, idx_v, pidx_v, rows_v,
                      sem):
        wid = lax.axis_index("s") * info.num_cores + lax.axis_index("c")
        base = wid * b_per_w
        pltpu.sync_copy(label_hbm.at[pl.ds(base, b_per_w)], idx_v)
        for h in range(b_per_w // 16):
            lv = idx_v[pl.ds(h * 16, 16)]
            pidx_v[pl.ds(h * 16, 16)] = (lax.div(lv, BLK) * (BLK // 2)
                                         + lax.rem(lv, BLK // 2))
        pltpu.async_copy(bank_hbm.at[pidx_v], rows_v, sem).wait()
        pltpu.sync_copy(rows_v, out_hbm.at[pl.ds(base, b_per_w)])

    return gather_kernel(label, bank2)


def _sweep_body(f_ref, bank_ref, s_ref, bank2_ref):
    i = pl.program_id(0)
    f = f_ref[...]
    blk = bank_ref[...]
    s_blk = lax.dot_general(f, blk, (((1,), (1,)), ((), ())),
                            preferred_element_type=jnp.float32) / T
    part = jnp.sum(jnp.exp(s_blk), axis=1, keepdims=True)
    bank2_ref[...] = jnp.concatenate(
        [blk[:BLK // 2, :], blk[BLK // 2:, :]], axis=1)

    @pl.when(i == 0)
    def _init():
        s_ref[...] = part

    @pl.when(i > 0)
    def _accum():
        s_ref[...] = s_ref[...] + part


def _combine_body(f_ref, old2_ref, lrow_ref, lcol_ref, s_ref, out_ref):
    f = f_ref[...]
    old2 = old2_ref[...]
    lcol = lcol_ref[...]
    odd = lax.div(lax.rem(lcol, BLK), BLK // 2).astype(jnp.float32)
    old = (old2[:, :CHANNEL] * (1.0 - odd)
           + old2[:, CHANNEL:] * odd)
    dn = (((1,), (1,)), ((), ()))
    p = lax.dot_general(f, old, dn, preferred_element_type=jnp.float32) / T
    q = lax.dot_general(f, f, dn, preferred_element_type=jnp.float32) / T
    lrow = lrow_ref[...]
    row_i = lax.broadcasted_iota(jnp.int32, (N, N), 0)
    col_i = lax.broadcasted_iota(jnp.int32, (N, N), 1)
    later_same = jnp.logical_and(lcol == lrow, row_i > col_i)
    winner = jnp.logical_not(jnp.any(later_same, axis=0, keepdims=True))
    wmask = jnp.broadcast_to(winner, (N, N))
    newv = ALPHA * p + q
    cmax = jnp.max(jnp.where(wmask, newv, -1e30), axis=1, keepdims=True)
    m_fin = jnp.maximum(cmax, 0.0)
    corr = jnp.sum(
        jnp.where(wmask, jnp.exp(newv - m_fin) - jnp.exp(p - m_fin), 0.0),
        axis=1, keepdims=True)
    total = s_ref[...] * jnp.exp(-m_fin) + corr
    lse = m_fin + jnp.log(total)
    pdiag = jnp.sum(jnp.where(row_i == col_i, p, 0.0), axis=1, keepdims=True)
    same_win = jnp.logical_and(lcol == lrow, wmask)
    qsel = jnp.sum(jnp.where(same_win, q, 0.0), axis=1, keepdims=True)
    pos = ALPHA * pdiag + qsel
    out_ref[...] = jnp.mean(lse - pos, axis=(0, 1), keepdims=True)


def kernel(f_normed, bank, label):
    lrow = label.reshape(1, N)
    lcol = label.reshape(N, 1)
    s_raw, bank2 = pl.pallas_call(
        _sweep_body,
        grid=(NBLK,),
        in_specs=[
            pl.BlockSpec((N, CHANNEL), lambda i: (0, 0)),
            pl.BlockSpec((BLK, CHANNEL), lambda i: (i, 0)),
        ],
        out_specs=[
            pl.BlockSpec((N, 1), lambda i: (0, 0)),
            pl.BlockSpec((BLK // 2, 2 * CHANNEL), lambda i: (i, 0)),
        ],
        out_shape=[
            jax.ShapeDtypeStruct((N, 1), jnp.float32),
            jax.ShapeDtypeStruct((ROWS2, 2 * CHANNEL), jnp.float32),
        ],
    )(f_normed, bank)
    old2 = _gather_pairs_sc(bank2, label)
    out = pl.pallas_call(
        _combine_body,
        in_specs=[
            pl.BlockSpec((N, CHANNEL), lambda: (0, 0)),
            pl.BlockSpec((N, 2 * CHANNEL), lambda: (0, 0)),
            pl.BlockSpec((1, N), lambda: (0, 0)),
            pl.BlockSpec((N, 1), lambda: (0, 0)),
            pl.BlockSpec((N, 1), lambda: (0, 0)),
        ],
        out_specs=pl.BlockSpec((1, 1), lambda: (0, 0)),
        out_shape=jax.ShapeDtypeStruct((1, 1), jnp.float32),
    )(f_normed, old2, lrow, lcol, s_raw)
    return out[0, 0]
